# Initial kernel scaffold; baseline (speedup 1.0000x reference)
import jax
import jax.numpy as jnp
from jax.experimental import pallas as pl


def kernel(x, Wr, W1, b1, W2, b2):
    raise NotImplementedError("write your pallas kernel here")


# speedup vs baseline: 1.3943x; 1.3943x over previous
import functools

import jax
import jax.numpy as jnp
from jax.experimental import pallas as pl
from jax.experimental.pallas import tpu as pltpu

N_TOKENS = 2048
D_MODEL = 768
D_FF = 1024
N_EXPERTS = 16
TOP_K = 2


def _router_kernel(x_ref, wr_ref, gate_ref):
    x = x_ref[...]
    wr = wr_ref[...]
    logits = jax.lax.dot_general(
        x, wr, (((1,), (1,)), ((), ())), preferred_element_type=jnp.float32)
    m = jnp.max(logits, axis=-1, keepdims=True)
    e = jnp.exp(logits - m)
    p = e / jnp.sum(e, axis=-1, keepdims=True)
    ids = jax.lax.broadcasted_iota(jnp.int32, p.shape, 1)
    a1 = jnp.argmax(p, axis=-1)
    oh1 = (ids == a1[:, None])
    w1 = jnp.sum(jnp.where(oh1, p, 0.0), axis=-1)
    p_masked = jnp.where(oh1, -1.0, p)
    a2 = jnp.argmax(p_masked, axis=-1)
    oh2 = (ids == a2[:, None])
    w2 = jnp.sum(jnp.where(oh2, p, 0.0), axis=-1)
    s = w1 + w2 + 1e-9
    gate = (jnp.where(oh1, w1[:, None], 0.0) + jnp.where(oh2, w2[:, None], 0.0))
    gate_ref[...] = gate / s[:, None]


def _expert_kernel(x_ref, gate_ref, w1_ref, b1_ref, w2_ref, b2_ref, out_ref):
    e = pl.program_id(0)
    x = x_ref[...]
    w1 = w1_ref[0]
    w2 = w2_ref[0]
    gate = gate_ref[...]
    col = jax.lax.broadcasted_iota(jnp.int32, gate.shape, 1)
    g = jnp.sum(jnp.where(col == e, gate, 0.0), axis=-1, keepdims=True)
    h = jax.lax.dot_general(x, w1, (((1,), (0,)), ((), ())),
                            preferred_element_type=jnp.float32)
    h = jnp.maximum(h + b1_ref[0], 0.0)
    y = jax.lax.dot_general(h, w2, (((1,), (0,)), ((), ())),
                            preferred_element_type=jnp.float32)
    contrib = g * (y + b2_ref[0])

    @pl.when(e == 0)
    def _init():
        out_ref[...] = contrib

    @pl.when(e > 0)
    def _acc():
        out_ref[...] += contrib


@jax.jit
def kernel(x, Wr, W1, b1, W2, b2):
    B, N, C = x.shape
    E, _, F = W1.shape
    x2 = x.reshape(N, C)

    gate = pl.pallas_call(
        _router_kernel,
        out_shape=jax.ShapeDtypeStruct((N, E), jnp.float32),
    )(x2, Wr)

    out = pl.pallas_call(
        _expert_kernel,
        grid=(E,),
        in_specs=[
            pl.BlockSpec((N, C), lambda e: (0, 0)),
            pl.BlockSpec((N, E), lambda e: (0, 0)),
            pl.BlockSpec((1, C, F), lambda e: (e, 0, 0)),
            pl.BlockSpec((1, 1, F), lambda e: (e, 0, 0)),
            pl.BlockSpec((1, F, C), lambda e: (e, 0, 0)),
            pl.BlockSpec((1, 1, C), lambda e: (e, 0, 0)),
        ],
        out_specs=pl.BlockSpec((N, C), lambda e: (0, 0)),
        out_shape=jax.ShapeDtypeStruct((N, C), jnp.float32),
        compiler_params=pltpu.CompilerParams(
            dimension_semantics=("arbitrary",),
        ),
    )(x2, gate, W1, b1.reshape(E, 1, F), W2, b2.reshape(E, 1, C))

    return out.reshape(B, N, C)
